# baseline (speedup 1.0000x reference)
import functools

import jax
import jax.numpy as jnp
from jax import lax
from jax.experimental import pallas as pl
from jax.experimental.pallas import tpu as pltpu
from jax.experimental.pallas import tpu_sc as plsc

D_MODEL = 128
NUM_ACTIONS = 10
GRID_SIZE = 64
COMBINED_ROWS = NUM_ACTIONS * GRID_SIZE * GRID_SIZE
PACKED_WORDS = D_MODEL // 2

NUM_CORES = 2
NUM_SUBCORES = 16
LANES = 16
NW = NUM_CORES * NUM_SUBCORES

GATHER_ROWS = 80
GPC = 8
SCHUNK = GATHER_ROWS * GPC

DEPTH = 3
NBUF = GPC
ONB = 4


def _combine_tables(action_table, x_table, y_table):
    def body(a_ref, x_ref, y_ref, o_ref):
        xr = x_ref[...]
        yr = y_ref[...]
        ar = a_ref[pl.ds(pl.program_id(0), 1), :]
        s = xr[:, None, :] + yr[None, :, :] + ar[0][None, None, :]
        o_ref[...] = s.astype(jnp.bfloat16)

    return pl.pallas_call(
        body,
        grid=(NUM_ACTIONS,),
        in_specs=[
            pl.BlockSpec((NUM_ACTIONS, D_MODEL), lambda a: (0, 0)),
            pl.BlockSpec((GRID_SIZE, D_MODEL), lambda a: (0, 0)),
            pl.BlockSpec((GRID_SIZE, D_MODEL), lambda a: (0, 0)),
        ],
        out_specs=pl.BlockSpec(
            (GRID_SIZE, GRID_SIZE, D_MODEL), lambda a: (a, 0, 0)
        ),
        out_shape=jax.ShapeDtypeStruct(
            (NUM_ACTIONS * GRID_SIZE, GRID_SIZE, D_MODEL), jnp.bfloat16
        ),
    )(action_table, x_table, y_table)


def _pack_rows(table_bf16):
    u = jax.lax.bitcast_convert_type(table_bf16, jnp.uint16)
    u = u.reshape(-1, 4, 2, 16)
    lo = u[:, :, 0, :].astype(jnp.uint32)
    hi = u[:, :, 1, :].astype(jnp.uint32)
    w = lo | (hi << 16)
    return jax.lax.bitcast_convert_type(w, jnp.int32).reshape(-1, PACKED_WORDS)


def _sc_lookup(n_tokens: int):
    assert n_tokens % (NW * SCHUNK) == 0
    per_w = n_tokens // NW
    n_schunks = per_w // SCHUNK
    mesh = plsc.VectorSubcoreMesh(
        core_axis_name="c", subcore_axis_name="s",
        num_cores=NUM_CORES, num_subcores=NUM_SUBCORES,
    )

    @functools.partial(
        pl.kernel,
        out_type=jax.ShapeDtypeStruct((n_tokens, D_MODEL), jnp.float32),
        mesh=mesh,
        compiler_params=pltpu.CompilerParams(use_tc_tiling_on_sc=False),
        scratch_types=[
            pltpu.VMEM((2, SCHUNK), jnp.int32),
            pltpu.VMEM((2, SCHUNK), jnp.int32),
            pltpu.VMEM((2, SCHUNK), jnp.int32),
            pltpu.VMEM((2, GPC, GATHER_ROWS), jnp.int32),
            pltpu.VMEM((NBUF, GATHER_ROWS, PACKED_WORDS), jnp.int32),
            pltpu.VMEM((ONB, GATHER_ROWS, D_MODEL), jnp.float32),
            pltpu.SemaphoreType.DMA,
            pltpu.SemaphoreType.DMA,
            pltpu.SemaphoreType.DMA,
        ],
    )
    def kern(a_hbm, x_hbm, y_hbm, axy_hbm, out_hbm,
             a_v, x_v, y_v, cidx_v, g_v, o_v, sem_g, sem_o, sem_i):
        wid = lax.axis_index("s") * NUM_CORES + lax.axis_index("c")
        wbase = wid * per_w

        def start_idx(slot, base):
            sl = pl.ds(base, SCHUNK)
            pltpu.async_copy(a_hbm.at[sl], a_v.at[slot], sem_i)
            pltpu.async_copy(x_hbm.at[sl], x_v.at[slot], sem_i)
            pltpu.async_copy(y_hbm.at[sl], y_v.at[slot], sem_i)

        def drain_idx(slot):
            dummy = pl.ds(0, SCHUNK)
            pltpu.make_async_copy(a_hbm.at[dummy], a_v.at[slot], sem_i).wait()
            pltpu.make_async_copy(x_hbm.at[dummy], x_v.at[slot], sem_i).wait()
            pltpu.make_async_copy(y_hbm.at[dummy], y_v.at[slot], sem_i).wait()

        def compute_cidx(slot):
            gpr = GATHER_ROWS // LANES
            for k in range(SCHUNK // LANES):
                g = pl.ds(k * LANES, LANES)
                av = jnp.clip(a_v[slot, g], 0, NUM_ACTIONS - 1)
                xv = jnp.clip(x_v[slot, g], 0, GRID_SIZE - 1)
                yv = jnp.clip(y_v[slot, g], 0, GRID_SIZE - 1)
                cidx_v[slot, k // gpr, pl.ds((k % gpr) * LANES, LANES)] = (
                    av * (GRID_SIZE * GRID_SIZE) + xv * GRID_SIZE + yv
                )

        def start_gather(slot, row, buf):
            pltpu.async_copy(
                axy_hbm.at[cidx_v.at[slot, row]], g_v.at[buf], sem_g
            )

        def drain_gather(buf):
            pltpu.make_async_copy(
                axy_hbm.at[pl.ds(0, GATHER_ROWS)], g_v.at[buf], sem_g
            ).wait()

        def start_out(obuf, base):
            pltpu.async_copy(
                o_v.at[obuf], out_hbm.at[pl.ds(base, GATHER_ROWS)], sem_o
            )

        def drain_out(obuf):
            pltpu.make_async_copy(
                o_v.at[obuf], out_hbm.at[pl.ds(wbase, GATHER_ROWS)], sem_o
            ).wait()

        def convert_chunk(buf, obuf):
            hi_mask = jnp.full((LANES,), -65536, dtype=jnp.int32)

            @plsc.parallel_loop(0, GATHER_ROWS, 1, unroll=8)
            def _(row):
                for m in range(4):
                    w = g_v[buf, row, pl.ds(m * LANES, LANES)]
                    lo = lax.bitcast_convert_type(w << 16, jnp.float32)
                    hi = lax.bitcast_convert_type(w & hi_mask, jnp.float32)
                    o_v[obuf, row, pl.ds(m * 2 * LANES, LANES)] = lo
                    o_v[obuf, row, pl.ds((m * 2 + 1) * LANES, LANES)] = hi

        start_idx(0, wbase)
        drain_idx(0)
        compute_cidx(0)
        start_idx(1, wbase + SCHUNK)
        for j0 in range(DEPTH):
            start_gather(0, j0, j0)

        def schunk_body(s, carry):
            p = lax.rem(s, 2)
            q = lax.rem(s + 1, 2)
            base = wbase + s * SCHUNK

            @pl.when(s + 1 < n_schunks)
            def _():
                drain_idx(q)
                compute_cidx(q)

                @pl.when(s + 2 < n_schunks)
                def _():
                    start_idx(p, base + 2 * SCHUNK)

            for j in range(GPC):
                gbuf = j % NBUF
                obuf = j % ONB
                drain_gather(gbuf)
                if j < GPC - DEPTH:
                    start_gather(p, j + DEPTH, (j + DEPTH) % NBUF)
                else:
                    @pl.when(s + 1 < n_schunks)
                    def _():
                        start_gather(q, (j + DEPTH) % GPC, (j + DEPTH) % NBUF)
                if j >= ONB:
                    drain_out(obuf)
                else:
                    @pl.when(s > 0)
                    def _():
                        drain_out(obuf)
                convert_chunk(gbuf, obuf)
                start_out(obuf, base + j * GATHER_ROWS)
            return carry

        lax.fori_loop(0, n_schunks, schunk_body, 0)
        for j0 in range(ONB):
            drain_out((GPC - ONB + j0) % ONB)

    return kern


def kernel(action_type, x, y, action_table, x_table, y_table):
    b, l = action_type.shape
    n = b * l
    a_flat = action_type.reshape(n).astype(jnp.int32)
    x_flat = x.reshape(n).astype(jnp.int32)
    y_flat = y.reshape(n).astype(jnp.int32)
    axy = _pack_rows(
        _combine_tables(action_table, x_table, y_table).reshape(
            COMBINED_ROWS, D_MODEL
        )
    )
    out = _sc_lookup(n)(a_flat, x_flat, y_flat, axy)
    return out.reshape(b, l, D_MODEL)


# speedup vs baseline: 1.0461x; 1.0461x over previous
import functools

import jax
import jax.numpy as jnp
from jax import lax
from jax.experimental import pallas as pl
from jax.experimental.pallas import tpu as pltpu
from jax.experimental.pallas import tpu_sc as plsc

D_MODEL = 128
NUM_ACTIONS = 10
GRID_SIZE = 64
COMBINED_ROWS = NUM_ACTIONS * GRID_SIZE * GRID_SIZE

NUM_CORES = 2
NUM_SUBCORES = 16
LANES = 16
NW = NUM_CORES * NUM_SUBCORES

GATHER_ROWS = 80
GPC = 8
SCHUNK = GATHER_ROWS * GPC

DEPTH = 3
NBUF = GPC
OUT_DEPTH = NBUF - DEPTH


def _combine_tables(action_table, x_table, y_table):
    def body(a_ref, x_ref, y_ref, o_ref):
        xr = x_ref[...]
        yr = y_ref[...]
        ar = a_ref[pl.ds(pl.program_id(0), 1), :]
        o_ref[...] = xr[:, None, :] + yr[None, :, :] + ar[0][None, None, :]

    return pl.pallas_call(
        body,
        grid=(NUM_ACTIONS,),
        in_specs=[
            pl.BlockSpec((NUM_ACTIONS, D_MODEL), lambda a: (0, 0)),
            pl.BlockSpec((GRID_SIZE, D_MODEL), lambda a: (0, 0)),
            pl.BlockSpec((GRID_SIZE, D_MODEL), lambda a: (0, 0)),
        ],
        out_specs=pl.BlockSpec(
            (GRID_SIZE, GRID_SIZE, D_MODEL), lambda a: (a, 0, 0)
        ),
        out_shape=jax.ShapeDtypeStruct(
            (NUM_ACTIONS * GRID_SIZE, GRID_SIZE, D_MODEL), jnp.float32
        ),
    )(action_table, x_table, y_table)


def _sc_lookup(n_tokens: int):
    assert n_tokens % (NW * SCHUNK) == 0
    per_w = n_tokens // NW
    n_schunks = per_w // SCHUNK
    mesh = plsc.VectorSubcoreMesh(
        core_axis_name="c", subcore_axis_name="s",
        num_cores=NUM_CORES, num_subcores=NUM_SUBCORES,
    )

    @functools.partial(
        pl.kernel,
        out_type=jax.ShapeDtypeStruct((n_tokens, D_MODEL), jnp.float32),
        mesh=mesh,
        scratch_types=[
            pltpu.VMEM((2, SCHUNK), jnp.int32),
            pltpu.VMEM((2, SCHUNK), jnp.int32),
            pltpu.VMEM((2, SCHUNK), jnp.int32),
            pltpu.VMEM((2, GPC, GATHER_ROWS), jnp.int32),
            pltpu.VMEM((NBUF, GATHER_ROWS, D_MODEL), jnp.float32),
            pltpu.SemaphoreType.DMA,
            pltpu.SemaphoreType.DMA,
            pltpu.SemaphoreType.DMA,
        ],
    )
    def kern(a_hbm, x_hbm, y_hbm, axy_hbm, out_hbm,
             a_v, x_v, y_v, cidx_v, rows_v, sem_g, sem_o, sem_i):
        wid = lax.axis_index("s") * NUM_CORES + lax.axis_index("c")
        wbase = wid * per_w

        def start_idx(slot, base):
            sl = pl.ds(base, SCHUNK)
            pltpu.async_copy(a_hbm.at[sl], a_v.at[slot], sem_i)
            pltpu.async_copy(x_hbm.at[sl], x_v.at[slot], sem_i)
            pltpu.async_copy(y_hbm.at[sl], y_v.at[slot], sem_i)

        def drain_idx(slot):
            dummy = pl.ds(0, SCHUNK)
            pltpu.make_async_copy(a_hbm.at[dummy], a_v.at[slot], sem_i).wait()
            pltpu.make_async_copy(x_hbm.at[dummy], x_v.at[slot], sem_i).wait()
            pltpu.make_async_copy(y_hbm.at[dummy], y_v.at[slot], sem_i).wait()

        def compute_cidx(slot):
            gpr = GATHER_ROWS // LANES
            for k in range(SCHUNK // LANES):
                g = pl.ds(k * LANES, LANES)
                av = jnp.clip(a_v[slot, g], 0, NUM_ACTIONS - 1)
                xv = jnp.clip(x_v[slot, g], 0, GRID_SIZE - 1)
                yv = jnp.clip(y_v[slot, g], 0, GRID_SIZE - 1)
                cidx_v[slot, k // gpr, pl.ds((k % gpr) * LANES, LANES)] = (
                    av * (GRID_SIZE * GRID_SIZE) + xv * GRID_SIZE + yv
                )

        def start_gather(slot, row, buf):
            pltpu.async_copy(
                axy_hbm.at[cidx_v.at[slot, row]], rows_v.at[buf], sem_g
            )

        def drain_gather(buf):
            pltpu.make_async_copy(
                axy_hbm.at[pl.ds(0, GATHER_ROWS)], rows_v.at[buf], sem_g
            ).wait()

        def start_out(buf, base):
            pltpu.async_copy(
                rows_v.at[buf], out_hbm.at[pl.ds(base, GATHER_ROWS)], sem_o
            )

        def drain_out(buf):
            pltpu.make_async_copy(
                rows_v.at[buf], out_hbm.at[pl.ds(wbase, GATHER_ROWS)], sem_o
            ).wait()

        start_idx(0, wbase)
        drain_idx(0)
        compute_cidx(0)
        start_idx(1, wbase + SCHUNK)
        for j0 in range(DEPTH):
            start_gather(0, j0, j0)

        def schunk_body(s, carry):
            p = lax.rem(s, 2)
            q = lax.rem(s + 1, 2)
            base = wbase + s * SCHUNK

            @pl.when(s + 1 < n_schunks)
            def _():
                drain_idx(q)
                compute_cidx(q)

                @pl.when(s + 2 < n_schunks)
                def _():
                    start_idx(p, base + 2 * SCHUNK)

            for j in range(GPC):
                buf = j % NBUF
                if j >= OUT_DEPTH:
                    drain_out((j - OUT_DEPTH) % NBUF)
                else:
                    @pl.when(s > 0)
                    def _():
                        drain_out((j - OUT_DEPTH) % NBUF)
                drain_gather(buf)
                if j < GPC - DEPTH:
                    start_gather(p, j + DEPTH, (j + DEPTH) % NBUF)
                else:
                    @pl.when(s + 1 < n_schunks)
                    def _():
                        start_gather(q, (j + DEPTH) % GPC, (j + DEPTH) % NBUF)
                start_out(buf, base + j * GATHER_ROWS)
            return carry

        lax.fori_loop(0, n_schunks, schunk_body, 0)
        for j0 in range(OUT_DEPTH):
            drain_out((GPC - OUT_DEPTH + j0) % NBUF)

    return kern


def kernel(action_type, x, y, action_table, x_table, y_table):
    b, l = action_type.shape
    n = b * l
    a_flat = action_type.reshape(n).astype(jnp.int32)
    x_flat = x.reshape(n).astype(jnp.int32)
    y_flat = y.reshape(n).astype(jnp.int32)
    axy = _combine_tables(action_table, x_table, y_table).reshape(
        COMBINED_ROWS, D_MODEL
    )
    out = _sc_lookup(n)(a_flat, x_flat, y_flat, axy)
    return out.reshape(b, l, D_MODEL)
